# baseline (speedup 1.0000x reference)
import jax
import jax.numpy as jnp
from jax import lax
from jax.experimental import pallas as pl
from jax.experimental.pallas import tpu as pltpu
from jax.experimental.pallas import tpu_sc as plsc

BATCH = 16384
HIST = 200
Z_N = 16
CHUNK = 128
NUM_WORKERS = 32
B_PER_W = BATCH // NUM_WORKERS
J_PER_W = B_PER_W // CHUNK


def _gather_body(x_hbm, mu_hbm, lv_hbm, out_z, out_mu, out_lv,
                 idx_v, rows_mu, rows_lv, t_mu, t_lv,
                 sem_g0, sem_g1, sem_o0, sem_o1, sem_i0, sem_i1):
    wid = lax.axis_index("s") * 2 + lax.axis_index("c")
    jb = wid * J_PER_W
    b0 = wid * B_PER_W
    sems_g = (sem_g0, sem_g1)
    sems_o = (sem_o0, sem_o1)
    sems_i = (sem_i0, sem_i1)

    def fire_idx(h, b):
        pltpu.async_copy(
            x_hbm.at[h // 8, pl.ds(jb, J_PER_W), h % 8], idx_v.at[b],
            sems_i[b])

    def fire(b):
        pltpu.make_async_copy(x_hbm.at[0, pl.ds(0, J_PER_W), 0],
                              idx_v.at[b], sems_i[b]).wait()
        for j in range(J_PER_W):
            pltpu.async_copy(mu_hbm.at[idx_v.at[b, j]],
                             rows_mu.at[b, pl.ds(j * CHUNK, CHUNK)],
                             sems_g[b])
            pltpu.async_copy(lv_hbm.at[idx_v.at[b, j]],
                             rows_lv.at[b, pl.ds(j * CHUNK, CHUNK)],
                             sems_g[b])

    def drain_gather(b):
        pltpu.make_async_copy(mu_hbm.at[pl.ds(0, B_PER_W)],
                              rows_mu.at[b], sems_g[b]).wait()
        pltpu.make_async_copy(lv_hbm.at[pl.ds(0, B_PER_W)],
                              rows_lv.at[b], sems_g[b]).wait()

    def transpose(b):
        cols = [jnp.full((16,), z, jnp.int32) for z in range(Z_N)]

        @plsc.parallel_loop(0, B_PER_W // 16)
        def jloop(j16):
            rbase = j16 * 16
            row_idx = rbase + lax.iota(jnp.int32, 16)
            vm = [plsc.load_gather(rows_mu.at[b], [row_idx, cols[z]])
                  for z in range(Z_N)]
            vl = [plsc.load_gather(rows_lv.at[b], [row_idx, cols[z]])
                  for z in range(Z_N)]
            for z in range(Z_N):
                t_mu[b, z, pl.ds(rbase, 16)] = vm[z]
                t_lv[b, z, pl.ds(rbase, 16)] = vl[z]

    def fire_out(h, b):
        pltpu.async_copy(t_mu.at[b], out_mu.at[h, :, pl.ds(b0, B_PER_W)],
                         sems_o[b])
        pltpu.async_copy(t_mu.at[b], out_z.at[h, :, pl.ds(b0, B_PER_W)],
                         sems_o[b])
        pltpu.async_copy(t_lv.at[b], out_lv.at[h, :, pl.ds(b0, B_PER_W)],
                         sems_o[b])

    def drain_out(b):
        pltpu.make_async_copy(t_mu.at[b], out_mu.at[0, :, pl.ds(b0, B_PER_W)],
                              sems_o[b]).wait()
        pltpu.make_async_copy(t_mu.at[b], out_z.at[0, :, pl.ds(b0, B_PER_W)],
                              sems_o[b]).wait()
        pltpu.make_async_copy(t_lv.at[b], out_lv.at[0, :, pl.ds(b0, B_PER_W)],
                              sems_o[b]).wait()

    fire_idx(0, 0)
    fire_idx(1, 1)
    fire(0)
    drain_gather(0)
    fire_idx(2, 0)
    fire(1)
    transpose(0)
    fire_out(0, 0)

    def outer(g, carry):
        h0 = 2 * g
        drain_out(0)
        fire(0)
        drain_gather(1)
        fire_idx(h0 + 1, 1)
        transpose(1)
        fire_out(h0 - 1, 1)
        h1 = 2 * g + 1
        drain_out(1)
        fire(1)
        drain_gather(0)

        @pl.when(h1 + 1 < HIST)
        def _():
            fire_idx(h1 + 1, 0)

        transpose(0)
        fire_out(h1 - 1, 0)
        return carry

    lax.fori_loop(1, HIST // 2, outer, 0)

    drain_gather(1)
    transpose(1)
    fire_out(HIST - 1, 1)
    drain_out(0)
    drain_out(1)


@jax.jit
def kernel(x, embed_mu, embed_logvar):
    x4 = jnp.transpose(
        x.astype(jnp.int32).T.reshape(HIST // 8, 8, BATCH // CHUNK, CHUNK),
        (0, 2, 1, 3))
    mesh = plsc.VectorSubcoreMesh(core_axis_name="c", subcore_axis_name="s")
    out_t = jax.ShapeDtypeStruct((HIST, Z_N, BATCH), jnp.float32)
    z_t, mu_t, lv_t = pl.kernel(
        _gather_body,
        out_type=[out_t, out_t, out_t],
        mesh=mesh,
        compiler_params=pltpu.CompilerParams(use_tc_tiling_on_sc=False,
                                              needs_layout_passes=False),
        scratch_types=[
            pltpu.VMEM((2, J_PER_W, CHUNK), jnp.int32),
            pltpu.VMEM((2, B_PER_W, Z_N), jnp.float32),
            pltpu.VMEM((2, B_PER_W, Z_N), jnp.float32),
            pltpu.VMEM((2, Z_N, B_PER_W), jnp.float32),
            pltpu.VMEM((2, Z_N, B_PER_W), jnp.float32),
            pltpu.SemaphoreType.DMA,
            pltpu.SemaphoreType.DMA,
            pltpu.SemaphoreType.DMA,
            pltpu.SemaphoreType.DMA,
            pltpu.SemaphoreType.DMA,
            pltpu.SemaphoreType.DMA,
        ],
    )(x4, embed_mu, embed_logvar)
    z = jnp.transpose(z_t, (2, 0, 1))
    mu = jnp.transpose(mu_t, (2, 0, 1))
    logvar = jnp.transpose(lv_t, (2, 0, 1))
    return (z, mu, logvar)
